# baseline (speedup 1.0000x reference)
import functools

import jax
import jax.numpy as jnp
from jax import lax
from jax.experimental import pallas as pl
from jax.experimental.pallas import tpu as pltpu
from jax.experimental.pallas import tpu_sc as plsc

D = 64
DP = 128
NUM_ENT = 1000000
NUM_REL = 1000
BATCH = 16384
NUM_WORKERS = 32
PER_W = BATCH // NUM_WORKERS
CHUNK = 128
LANES = 16
NBLK = PER_W // CHUNK
ROCT = D // 8
ENT_N = 2 * BATCH
OCT_W = 8 * CHUNK
TBLK = 2048
TGRID = (NUM_ENT + TBLK - 1) // TBLK


def _transpose_block(tt_ref, out_ref):
    x = tt_ref[...]
    eye = (lax.broadcasted_iota(jnp.int32, (D, D), 0)
           == lax.broadcasted_iota(jnp.int32, (D, D), 1)).astype(jnp.float32)
    xt = lax.dot_general(x, eye, (((0,), (0,)), ((), ())),
                         preferred_element_type=jnp.float32)
    xt3 = xt.reshape(TBLK // 2, 2, D)
    out_ref[:, 0:D] = xt3[:, 0, :]
    out_ref[:, D:DP] = xt3[:, 1, :]


_pack_rows = pl.pallas_call(
    _transpose_block,
    grid=(TGRID,),
    in_specs=[pl.BlockSpec((D, TBLK), lambda k: (0, k))],
    out_specs=pl.BlockSpec((TBLK // 2, DP), lambda k: (k, 0)),
    out_shape=jax.ShapeDtypeStruct((NUM_ENT // 2, DP), jnp.float32),
)


def _sc_body(eidx_hbm, ridx_hbm, tview_hbm, rtab_hbm,
             ent_out_hbm, rel_out_hbm,
             eidx_v, epair_v, ridx_v, rows_v, oct_v, rtab_v,
             sem, sem2, sem3):
    wid = lax.axis_index("s") * 2 + lax.axis_index("c")
    base = wid * PER_W

    rtab_copy = pltpu.async_copy(rtab_hbm, rtab_v, sem2)
    pltpu.sync_copy(eidx_hbm.at[pl.ds(base, PER_W)],
                    eidx_v.at[pl.ds(0, PER_W)])
    pltpu.sync_copy(eidx_hbm.at[pl.ds(BATCH + base, PER_W)],
                    eidx_v.at[pl.ds(PER_W, PER_W)])
    pltpu.sync_copy(ridx_hbm.at[pl.ds(base, PER_W)], ridx_v)

    lane_iota = lax.iota(jnp.int32, LANES)

    def ent_chunk(c, carry):
        j = c // NBLK
        ib = wid * NBLK + (c % NBLK)
        for g in range(CHUNK // LANES):
            evec = eidx_v[pl.ds(c * CHUNK + g * LANES, LANES)]
            epair_v[pl.ds(g * LANES, LANES)] = evec >> 1
        pltpu.async_copy(tview_hbm.at[epair_v], rows_v, sem).wait()

        def per_oct(r, carry2):
            rb = r & 1

            @pl.when(c * ROCT + r > 1)
            def _():
                pltpu.make_async_copy(oct_v.at[pl.ds(0, OCT_W)],
                                      ent_out_hbm.at[pl.ds(0, OCT_W)],
                                      sem3).wait()
            for g in range(CHUNK // LANES):
                evec = eidx_v[pl.ds(c * CHUNK + g * LANES, LANES)]
                rowvec = g * LANES + lane_iota
                colbase = (evec & 1) * D + r * 8
                for dr in range(8):
                    vals = plsc.load_gather(rows_v, [rowvec, colbase + dr])
                    oct_v[pl.ds(rb * OCT_W + dr * CHUNK + g * LANES,
                                LANES)] = vals
            pltpu.async_copy(
                oct_v.at[pl.ds(rb * OCT_W, OCT_W)],
                ent_out_hbm.at[pl.ds(((j * ROCT + r) * (BATCH // CHUNK)
                                      + ib) * OCT_W, OCT_W)], sem3)
            return carry2

        lax.fori_loop(0, ROCT, per_oct, 0, unroll=False)
        return carry

    lax.fori_loop(0, 2 * NBLK, ent_chunk, 0, unroll=False)
    for _ in range(2):
        pltpu.make_async_copy(oct_v.at[pl.ds(0, OCT_W)],
                              ent_out_hbm.at[pl.ds(0, OCT_W)], sem3).wait()

    rtab_copy.wait()

    def rel_blk(b, carry):
        ib = wid * NBLK + b

        def per_oct(r, carry2):
            rb = r & 1

            @pl.when(b * ROCT + r > 1)
            def _():
                pltpu.make_async_copy(oct_v.at[pl.ds(0, OCT_W)],
                                      rel_out_hbm.at[pl.ds(0, OCT_W)],
                                      sem3).wait()
            for g in range(CHUNK // LANES):
                rvec = ridx_v[pl.ds(b * CHUNK + g * LANES, LANES)]
                for dr in range(8):
                    vals = plsc.load_gather(
                        rtab_v, [(r * 8 + dr) * NUM_REL + rvec])
                    oct_v[pl.ds(rb * OCT_W + dr * CHUNK + g * LANES,
                                LANES)] = vals
            pltpu.async_copy(
                oct_v.at[pl.ds(rb * OCT_W, OCT_W)],
                rel_out_hbm.at[pl.ds((r * (BATCH // CHUNK) + ib) * OCT_W,
                                     OCT_W)], sem3)
            return carry2

        lax.fori_loop(0, ROCT, per_oct, 0, unroll=False)
        return carry

    lax.fori_loop(0, NBLK, rel_blk, 0, unroll=False)
    for _ in range(2):
        pltpu.make_async_copy(oct_v.at[pl.ds(0, OCT_W)],
                              rel_out_hbm.at[pl.ds(0, OCT_W)], sem3).wait()


@functools.partial(
    pl.kernel,
    out_type=(
        jax.ShapeDtypeStruct((2 * D * BATCH,), jnp.float32),
        jax.ShapeDtypeStruct((D * BATCH,), jnp.float32),
    ),
    mesh=plsc.VectorSubcoreMesh(core_axis_name="c", subcore_axis_name="s"),
    compiler_params=pltpu.CompilerParams(use_tc_tiling_on_sc=False,
                                         needs_layout_passes=False),
    scratch_types=[
        pltpu.VMEM((2 * PER_W,), jnp.int32),
        pltpu.VMEM((CHUNK,), jnp.int32),
        pltpu.VMEM((PER_W,), jnp.int32),
        pltpu.VMEM((CHUNK, DP), jnp.float32),
        pltpu.VMEM((2 * OCT_W,), jnp.float32),
        pltpu.VMEM((D * NUM_REL,), jnp.float32),
        pltpu.SemaphoreType.DMA,
        pltpu.SemaphoreType.DMA,
        pltpu.SemaphoreType.DMA,
    ],
)
def _sc_lookup(eidx, ridx, tview, rtab, ent_out, rel_out,
               eidx_v, epair_v, ridx_v, rows_v, oct_v, rtab_v,
               sem, sem2, sem3):
    _sc_body(eidx, ridx, tview, rtab, ent_out, rel_out,
             eidx_v, epair_v, ridx_v, rows_v, oct_v, rtab_v,
             sem, sem2, sem3)


def kernel(idx, rel, ent_table, rel_table):
    eidx = jnp.transpose(idx.reshape(BATCH, 2)).reshape(ENT_N)
    eidx = eidx.astype(jnp.int32)
    ridx = rel.astype(jnp.int32)
    tview = _pack_rows(jnp.transpose(ent_table))
    rtab = jnp.transpose(rel_table).reshape(D * NUM_REL)
    ent_o, rel_o = _sc_lookup(eidx, ridx, tview, rtab)
    ent_emb = jnp.transpose(
        ent_o.reshape(2, ROCT, BATCH // CHUNK, 8, CHUNK),
        (2, 4, 0, 1, 3)).reshape(BATCH, 1, 2, D)
    rel_emb = jnp.transpose(
        rel_o.reshape(ROCT, BATCH // CHUNK, 8, CHUNK),
        (1, 3, 0, 2)).reshape(BATCH, D)
    return (ent_emb, rel_emb)


# speedup vs baseline: 1.2025x; 1.2025x over previous
import functools

import jax
import jax.numpy as jnp
from jax import lax
from jax.experimental import pallas as pl
from jax.experimental.pallas import tpu as pltpu
from jax.experimental.pallas import tpu_sc as plsc

D = 64
DP = 128
NUM_ENT = 1000000
NUM_REL = 1000
BATCH = 16384
NUM_WORKERS = 32
PER_W = BATCH // NUM_WORKERS
CHUNK = 128
LANES = 16
NBLK = PER_W // CHUNK
ROCT = D // 8
ENT_N = 2 * BATCH
OCT_W = 8 * CHUNK
TBLK = 1024
TGRID = 489
KP = TGRID * TBLK


def _transpose_block(lo_ref, hi_ref, out_ref):
    eye = (lax.broadcasted_iota(jnp.int32, (D, D), 0)
           == lax.broadcasted_iota(jnp.int32, (D, D), 1)).astype(jnp.float32)
    out_ref[:, 0:D] = lax.dot_general(
        lo_ref[...], eye, (((0,), (0,)), ((), ())),
        preferred_element_type=jnp.float32)
    out_ref[:, D:DP] = lax.dot_general(
        hi_ref[...], eye, (((0,), (0,)), ((), ())),
        preferred_element_type=jnp.float32)


_pack_rows = pl.pallas_call(
    _transpose_block,
    grid=(TGRID,),
    in_specs=[pl.BlockSpec((D, TBLK), lambda k: (0, k)),
              pl.BlockSpec((D, TBLK),
                           lambda k: (0, jnp.minimum(k + TGRID,
                                                     NUM_ENT // TBLK)))],
    out_specs=pl.BlockSpec((TBLK, DP), lambda k: (k, 0)),
    out_shape=jax.ShapeDtypeStruct((KP, DP), jnp.float32),
)


def _sc_body(eidx_hbm, ridx_hbm, tview_hbm, rtab_hbm,
             ent_out_hbm, rel_out_hbm,
             eidx_v, epair_v, ridx_v, rows_v, oct_v, rtab_v,
             sem, sem2, sem3):
    wid = lax.axis_index("s") * 2 + lax.axis_index("c")
    base = wid * PER_W

    rtab_copy = pltpu.async_copy(rtab_hbm, rtab_v, sem2)
    pltpu.sync_copy(eidx_hbm.at[pl.ds(base, PER_W)],
                    eidx_v.at[pl.ds(0, PER_W)])
    pltpu.sync_copy(eidx_hbm.at[pl.ds(BATCH + base, PER_W)],
                    eidx_v.at[pl.ds(PER_W, PER_W)])
    pltpu.sync_copy(ridx_hbm.at[pl.ds(base, PER_W)], ridx_v)

    lane_iota = lax.iota(jnp.int32, LANES)

    def ent_chunk(c, carry):
        j = c // NBLK
        ib = wid * NBLK + (c % NBLK)
        for g in range(CHUNK // LANES):
            evec = eidx_v[pl.ds(c * CHUNK + g * LANES, LANES)]
            hvec = jnp.where(evec >= KP, 1, 0)
            epair_v[pl.ds(g * LANES, LANES)] = evec - hvec * KP
        pltpu.async_copy(tview_hbm.at[epair_v], rows_v, sem).wait()

        def per_oct(r, carry2):
            rb = r & 1

            @pl.when(c * ROCT + r > 1)
            def _():
                pltpu.make_async_copy(oct_v.at[pl.ds(0, OCT_W)],
                                      ent_out_hbm.at[pl.ds(0, OCT_W)],
                                      sem3).wait()
            for g in range(CHUNK // LANES):
                evec = eidx_v[pl.ds(c * CHUNK + g * LANES, LANES)]
                rowvec = g * LANES + lane_iota
                colbase = jnp.where(evec >= KP, D, 0) + r * 8
                for dr in range(8):
                    vals = plsc.load_gather(rows_v, [rowvec, colbase + dr])
                    oct_v[pl.ds(rb * OCT_W + dr * CHUNK + g * LANES,
                                LANES)] = vals
            pltpu.async_copy(
                oct_v.at[pl.ds(rb * OCT_W, OCT_W)],
                ent_out_hbm.at[pl.ds(((j * ROCT + r) * (BATCH // CHUNK)
                                      + ib) * OCT_W, OCT_W)], sem3)
            return carry2

        lax.fori_loop(0, ROCT, per_oct, 0, unroll=False)
        return carry

    lax.fori_loop(0, 2 * NBLK, ent_chunk, 0, unroll=False)
    for _ in range(2):
        pltpu.make_async_copy(oct_v.at[pl.ds(0, OCT_W)],
                              ent_out_hbm.at[pl.ds(0, OCT_W)], sem3).wait()

    rtab_copy.wait()

    def rel_blk(b, carry):
        ib = wid * NBLK + b

        def per_oct(r, carry2):
            rb = r & 1

            @pl.when(b * ROCT + r > 1)
            def _():
                pltpu.make_async_copy(oct_v.at[pl.ds(0, OCT_W)],
                                      rel_out_hbm.at[pl.ds(0, OCT_W)],
                                      sem3).wait()
            for g in range(CHUNK // LANES):
                rvec = ridx_v[pl.ds(b * CHUNK + g * LANES, LANES)]
                for dr in range(8):
                    vals = plsc.load_gather(
                        rtab_v, [(r * 8 + dr) * NUM_REL + rvec])
                    oct_v[pl.ds(rb * OCT_W + dr * CHUNK + g * LANES,
                                LANES)] = vals
            pltpu.async_copy(
                oct_v.at[pl.ds(rb * OCT_W, OCT_W)],
                rel_out_hbm.at[pl.ds((r * (BATCH // CHUNK) + ib) * OCT_W,
                                     OCT_W)], sem3)
            return carry2

        lax.fori_loop(0, ROCT, per_oct, 0, unroll=False)
        return carry

    lax.fori_loop(0, NBLK, rel_blk, 0, unroll=False)
    for _ in range(2):
        pltpu.make_async_copy(oct_v.at[pl.ds(0, OCT_W)],
                              rel_out_hbm.at[pl.ds(0, OCT_W)], sem3).wait()


@functools.partial(
    pl.kernel,
    out_type=(
        jax.ShapeDtypeStruct((2 * D * BATCH,), jnp.float32),
        jax.ShapeDtypeStruct((D * BATCH,), jnp.float32),
    ),
    mesh=plsc.VectorSubcoreMesh(core_axis_name="c", subcore_axis_name="s"),
    compiler_params=pltpu.CompilerParams(use_tc_tiling_on_sc=False,
                                         needs_layout_passes=False),
    scratch_types=[
        pltpu.VMEM((2 * PER_W,), jnp.int32),
        pltpu.VMEM((CHUNK,), jnp.int32),
        pltpu.VMEM((PER_W,), jnp.int32),
        pltpu.VMEM((CHUNK, DP), jnp.float32),
        pltpu.VMEM((2 * OCT_W,), jnp.float32),
        pltpu.VMEM((D * NUM_REL,), jnp.float32),
        pltpu.SemaphoreType.DMA,
        pltpu.SemaphoreType.DMA,
        pltpu.SemaphoreType.DMA,
    ],
)
def _sc_lookup(eidx, ridx, tview, rtab, ent_out, rel_out,
               eidx_v, epair_v, ridx_v, rows_v, oct_v, rtab_v,
               sem, sem2, sem3):
    _sc_body(eidx, ridx, tview, rtab, ent_out, rel_out,
             eidx_v, epair_v, ridx_v, rows_v, oct_v, rtab_v,
             sem, sem2, sem3)


def kernel(idx, rel, ent_table, rel_table):
    eidx = jnp.transpose(idx.reshape(BATCH, 2)).reshape(ENT_N)
    eidx = eidx.astype(jnp.int32)
    ridx = rel.astype(jnp.int32)
    ttab = jnp.transpose(ent_table)
    tview = _pack_rows(ttab, ttab)
    rtab = jnp.transpose(rel_table).reshape(D * NUM_REL)
    ent_o, rel_o = _sc_lookup(eidx, ridx, tview, rtab)
    ent_emb = jnp.transpose(
        ent_o.reshape(2, ROCT, BATCH // CHUNK, 8, CHUNK),
        (2, 4, 0, 1, 3)).reshape(BATCH, 1, 2, D)
    rel_emb = jnp.transpose(
        rel_o.reshape(ROCT, BATCH // CHUNK, 8, CHUNK),
        (1, 3, 0, 2)).reshape(BATCH, D)
    return (ent_emb, rel_emb)


# speedup vs baseline: 1.5807x; 1.3145x over previous
import functools

import jax
import jax.numpy as jnp
from jax import lax
from jax.experimental import pallas as pl
from jax.experimental.pallas import tpu as pltpu
from jax.experimental.pallas import tpu_sc as plsc

D = 64
DP = 128
NUM_ENT = 1000000
NUM_REL = 1000
BATCH = 16384
NUM_WORKERS = 32
PER_W = BATCH // NUM_WORKERS
CHUNK = 128
LANES = 16
NBLK = PER_W // CHUNK
ROCT = D // 8
ENT_N = 2 * BATCH
OCT_W = 8 * CHUNK
TBLK = 2048
TGRID = 245
KP = TGRID * TBLK


def _transpose_block(lo_ref, hi_ref, out_ref):
    eye = (lax.broadcasted_iota(jnp.int32, (D, D), 0)
           == lax.broadcasted_iota(jnp.int32, (D, D), 1)).astype(jnp.float32)
    out_ref[:, 0:D] = lax.dot_general(
        lo_ref[...], eye, (((0,), (0,)), ((), ())),
        preferred_element_type=jnp.float32)
    out_ref[:, D:DP] = lax.dot_general(
        hi_ref[...], eye, (((0,), (0,)), ((), ())),
        preferred_element_type=jnp.float32)


_pack_rows = pl.pallas_call(
    _transpose_block,
    grid=(TGRID,),
    in_specs=[pl.BlockSpec((D, TBLK), lambda k: (0, k)),
              pl.BlockSpec((D, TBLK),
                           lambda k: (0, jnp.minimum(k + TGRID,
                                                     NUM_ENT // TBLK)))],
    out_specs=pl.BlockSpec((TBLK, DP), lambda k: (k, 0)),
    out_shape=jax.ShapeDtypeStruct((KP, DP), jnp.float32),
)


def _sc_body(eidx_hbm, ridx_hbm, tview_hbm, rtab_hbm,
             ent_out_hbm, rel_out_hbm,
             eidx_v, epair_v, ridx_v, rows_v, oct_v, rtab_v,
             sem, sem2, sem3):
    wid = lax.axis_index("s") * 2 + lax.axis_index("c")
    base = wid * PER_W

    rtab_copy = pltpu.async_copy(rtab_hbm, rtab_v, sem2)
    pltpu.sync_copy(eidx_hbm.at[pl.ds(base, PER_W)],
                    eidx_v.at[pl.ds(0, PER_W)])
    pltpu.sync_copy(eidx_hbm.at[pl.ds(BATCH + base, PER_W)],
                    eidx_v.at[pl.ds(PER_W, PER_W)])
    pltpu.sync_copy(ridx_hbm.at[pl.ds(base, PER_W)], ridx_v)

    lane_iota = lax.iota(jnp.int32, LANES)

    def ent_chunk(c, carry):
        j = c // NBLK
        ib = wid * NBLK + (c % NBLK)
        for g in range(CHUNK // LANES):
            evec = eidx_v[pl.ds(c * CHUNK + g * LANES, LANES)]
            hvec = jnp.where(evec >= KP, 1, 0)
            epair_v[pl.ds(g * LANES, LANES)] = evec - hvec * KP
        pltpu.async_copy(tview_hbm.at[epair_v], rows_v, sem).wait()

        def per_oct(r, carry2):
            rb = r & 1

            @pl.when(c * ROCT + r > 1)
            def _():
                pltpu.make_async_copy(oct_v.at[pl.ds(0, OCT_W)],
                                      ent_out_hbm.at[pl.ds(0, OCT_W)],
                                      sem3).wait()
            for g in range(CHUNK // LANES):
                evec = eidx_v[pl.ds(c * CHUNK + g * LANES, LANES)]
                rowvec = g * LANES + lane_iota
                colbase = jnp.where(evec >= KP, D, 0) + r * 8
                for dr in range(8):
                    vals = plsc.load_gather(rows_v, [rowvec, colbase + dr])
                    oct_v[pl.ds(rb * OCT_W + dr * CHUNK + g * LANES,
                                LANES)] = vals
            pltpu.async_copy(
                oct_v.at[pl.ds(rb * OCT_W, OCT_W)],
                ent_out_hbm.at[pl.ds(((j * ROCT + r) * (BATCH // CHUNK)
                                      + ib) * OCT_W, OCT_W)], sem3)
            return carry2

        lax.fori_loop(0, ROCT, per_oct, 0, unroll=False)
        return carry

    lax.fori_loop(0, 2 * NBLK, ent_chunk, 0, unroll=False)
    for _ in range(2):
        pltpu.make_async_copy(oct_v.at[pl.ds(0, OCT_W)],
                              ent_out_hbm.at[pl.ds(0, OCT_W)], sem3).wait()

    rtab_copy.wait()

    def rel_blk(b, carry):
        ib = wid * NBLK + b

        def per_oct(r, carry2):
            rb = r & 1

            @pl.when(b * ROCT + r > 1)
            def _():
                pltpu.make_async_copy(oct_v.at[pl.ds(0, OCT_W)],
                                      rel_out_hbm.at[pl.ds(0, OCT_W)],
                                      sem3).wait()
            for g in range(CHUNK // LANES):
                rvec = ridx_v[pl.ds(b * CHUNK + g * LANES, LANES)]
                for dr in range(8):
                    vals = plsc.load_gather(
                        rtab_v, [(r * 8 + dr) * NUM_REL + rvec])
                    oct_v[pl.ds(rb * OCT_W + dr * CHUNK + g * LANES,
                                LANES)] = vals
            pltpu.async_copy(
                oct_v.at[pl.ds(rb * OCT_W, OCT_W)],
                rel_out_hbm.at[pl.ds((r * (BATCH // CHUNK) + ib) * OCT_W,
                                     OCT_W)], sem3)
            return carry2

        lax.fori_loop(0, ROCT, per_oct, 0, unroll=False)
        return carry

    lax.fori_loop(0, NBLK, rel_blk, 0, unroll=False)
    for _ in range(2):
        pltpu.make_async_copy(oct_v.at[pl.ds(0, OCT_W)],
                              rel_out_hbm.at[pl.ds(0, OCT_W)], sem3).wait()


@functools.partial(
    pl.kernel,
    out_type=(
        jax.ShapeDtypeStruct((2 * D * BATCH,), jnp.float32),
        jax.ShapeDtypeStruct((D * BATCH,), jnp.float32),
    ),
    mesh=plsc.VectorSubcoreMesh(core_axis_name="c", subcore_axis_name="s"),
    compiler_params=pltpu.CompilerParams(use_tc_tiling_on_sc=False,
                                         needs_layout_passes=False),
    scratch_types=[
        pltpu.VMEM((2 * PER_W,), jnp.int32),
        pltpu.VMEM((CHUNK,), jnp.int32),
        pltpu.VMEM((PER_W,), jnp.int32),
        pltpu.VMEM((CHUNK, DP), jnp.float32),
        pltpu.VMEM((2 * OCT_W,), jnp.float32),
        pltpu.VMEM((D * NUM_REL,), jnp.float32),
        pltpu.SemaphoreType.DMA,
        pltpu.SemaphoreType.DMA,
        pltpu.SemaphoreType.DMA,
    ],
)
def _sc_lookup(eidx, ridx, tview, rtab, ent_out, rel_out,
               eidx_v, epair_v, ridx_v, rows_v, oct_v, rtab_v,
               sem, sem2, sem3):
    _sc_body(eidx, ridx, tview, rtab, ent_out, rel_out,
             eidx_v, epair_v, ridx_v, rows_v, oct_v, rtab_v,
             sem, sem2, sem3)


def kernel(idx, rel, ent_table, rel_table):
    eidx = jnp.transpose(idx.reshape(BATCH, 2)).reshape(ENT_N)
    eidx = eidx.astype(jnp.int32)
    ridx = rel.astype(jnp.int32)
    ttab = jnp.transpose(ent_table)
    tview = _pack_rows(ttab, ttab)
    rtab = jnp.transpose(rel_table).reshape(D * NUM_REL)
    ent_o, rel_o = _sc_lookup(eidx, ridx, tview, rtab)
    ent_emb = jnp.transpose(
        ent_o.reshape(2, ROCT, BATCH // CHUNK, 8, CHUNK),
        (2, 4, 0, 1, 3)).reshape(BATCH, 1, 2, D)
    rel_emb = jnp.transpose(
        rel_o.reshape(ROCT, BATCH // CHUNK, 8, CHUNK),
        (1, 3, 0, 2)).reshape(BATCH, D)
    return (ent_emb, rel_emb)


# speedup vs baseline: 2.0617x; 1.3043x over previous
import functools

import jax
import jax.numpy as jnp
from jax import lax
from jax.experimental import pallas as pl
from jax.experimental.pallas import tpu as pltpu
from jax.experimental.pallas import tpu_sc as plsc

D = 64
DP = 128
NUM_ENT = 1000000
NUM_REL = 1000
BATCH = 16384
NUM_WORKERS = 32
PER_W = BATCH // NUM_WORKERS
CHUNK = 128
LANES = 16
NBLK = PER_W // CHUNK
ROCT = D // 8
ENT_N = 2 * BATCH
OCT_W = 8 * CHUNK
TBLK = 8192
TGRID = 62
KP = TGRID * TBLK


def _transpose_block(lo_ref, hi_ref, out_ref):
    eye = (lax.broadcasted_iota(jnp.int32, (D, D), 0)
           == lax.broadcasted_iota(jnp.int32, (D, D), 1)).astype(jnp.float32)
    out_ref[:, 0:D] = lax.dot_general(
        lo_ref[...], eye, (((0,), (0,)), ((), ())),
        preferred_element_type=jnp.float32)
    out_ref[:, D:DP] = lax.dot_general(
        hi_ref[...], eye, (((0,), (0,)), ((), ())),
        preferred_element_type=jnp.float32)


_pack_rows = pl.pallas_call(
    _transpose_block,
    grid=(TGRID,),
    in_specs=[pl.BlockSpec((D, TBLK), lambda k: (0, k)),
              pl.BlockSpec((D, TBLK),
                           lambda k: (0, jnp.minimum(k + TGRID,
                                                     NUM_ENT // TBLK)))],
    out_specs=pl.BlockSpec((TBLK, DP), lambda k: (k, 0)),
    out_shape=jax.ShapeDtypeStruct((KP, DP), jnp.float32),
)


def _sc_body(eidx_hbm, ridx_hbm, tview_hbm, rtab_hbm,
             ent_out_hbm, rel_out_hbm,
             eidx_v, epair_v, ridx_v, rows_v, oct_v, rtab_v,
             sem, sem2, sem3):
    wid = lax.axis_index("s") * 2 + lax.axis_index("c")
    base = wid * PER_W

    rtab_copy = pltpu.async_copy(rtab_hbm, rtab_v, sem2)
    pltpu.sync_copy(eidx_hbm.at[pl.ds(base, PER_W)],
                    eidx_v.at[pl.ds(0, PER_W)])
    pltpu.sync_copy(eidx_hbm.at[pl.ds(BATCH + base, PER_W)],
                    eidx_v.at[pl.ds(PER_W, PER_W)])
    pltpu.sync_copy(ridx_hbm.at[pl.ds(base, PER_W)], ridx_v)

    lane_iota = lax.iota(jnp.int32, LANES)

    def ent_chunk(c, carry):
        j = c // NBLK
        ib = wid * NBLK + (c % NBLK)
        for g in range(CHUNK // LANES):
            evec = eidx_v[pl.ds(c * CHUNK + g * LANES, LANES)]
            hvec = jnp.where(evec >= KP, 1, 0)
            epair_v[pl.ds(g * LANES, LANES)] = evec - hvec * KP
        pltpu.async_copy(tview_hbm.at[epair_v], rows_v, sem).wait()

        def per_oct(r, carry2):
            rb = r & 1

            @pl.when(c * ROCT + r > 1)
            def _():
                pltpu.make_async_copy(oct_v.at[pl.ds(0, OCT_W)],
                                      ent_out_hbm.at[pl.ds(0, OCT_W)],
                                      sem3).wait()
            for g in range(CHUNK // LANES):
                evec = eidx_v[pl.ds(c * CHUNK + g * LANES, LANES)]
                rowvec = g * LANES + lane_iota
                colbase = jnp.where(evec >= KP, D, 0) + r * 8
                for dr in range(8):
                    vals = plsc.load_gather(rows_v, [rowvec, colbase + dr])
                    oct_v[pl.ds(rb * OCT_W + dr * CHUNK + g * LANES,
                                LANES)] = vals
            pltpu.async_copy(
                oct_v.at[pl.ds(rb * OCT_W, OCT_W)],
                ent_out_hbm.at[pl.ds(((j * ROCT + r) * (BATCH // CHUNK)
                                      + ib) * OCT_W, OCT_W)], sem3)
            return carry2

        lax.fori_loop(0, ROCT, per_oct, 0, unroll=False)
        return carry

    lax.fori_loop(0, 2 * NBLK, ent_chunk, 0, unroll=False)
    for _ in range(2):
        pltpu.make_async_copy(oct_v.at[pl.ds(0, OCT_W)],
                              ent_out_hbm.at[pl.ds(0, OCT_W)], sem3).wait()

    rtab_copy.wait()

    def rel_blk(b, carry):
        ib = wid * NBLK + b

        def per_oct(r, carry2):
            rb = r & 1

            @pl.when(b * ROCT + r > 1)
            def _():
                pltpu.make_async_copy(oct_v.at[pl.ds(0, OCT_W)],
                                      rel_out_hbm.at[pl.ds(0, OCT_W)],
                                      sem3).wait()
            for g in range(CHUNK // LANES):
                rvec = ridx_v[pl.ds(b * CHUNK + g * LANES, LANES)]
                for dr in range(8):
                    vals = plsc.load_gather(
                        rtab_v, [(r * 8 + dr) * NUM_REL + rvec])
                    oct_v[pl.ds(rb * OCT_W + dr * CHUNK + g * LANES,
                                LANES)] = vals
            pltpu.async_copy(
                oct_v.at[pl.ds(rb * OCT_W, OCT_W)],
                rel_out_hbm.at[pl.ds((r * (BATCH // CHUNK) + ib) * OCT_W,
                                     OCT_W)], sem3)
            return carry2

        lax.fori_loop(0, ROCT, per_oct, 0, unroll=False)
        return carry

    lax.fori_loop(0, NBLK, rel_blk, 0, unroll=False)
    for _ in range(2):
        pltpu.make_async_copy(oct_v.at[pl.ds(0, OCT_W)],
                              rel_out_hbm.at[pl.ds(0, OCT_W)], sem3).wait()


@functools.partial(
    pl.kernel,
    out_type=(
        jax.ShapeDtypeStruct((2 * D * BATCH,), jnp.float32),
        jax.ShapeDtypeStruct((D * BATCH,), jnp.float32),
    ),
    mesh=plsc.VectorSubcoreMesh(core_axis_name="c", subcore_axis_name="s"),
    compiler_params=pltpu.CompilerParams(use_tc_tiling_on_sc=False,
                                         needs_layout_passes=False),
    scratch_types=[
        pltpu.VMEM((2 * PER_W,), jnp.int32),
        pltpu.VMEM((CHUNK,), jnp.int32),
        pltpu.VMEM((PER_W,), jnp.int32),
        pltpu.VMEM((CHUNK, DP), jnp.float32),
        pltpu.VMEM((2 * OCT_W,), jnp.float32),
        pltpu.VMEM((D * NUM_REL,), jnp.float32),
        pltpu.SemaphoreType.DMA,
        pltpu.SemaphoreType.DMA,
        pltpu.SemaphoreType.DMA,
    ],
)
def _sc_lookup(eidx, ridx, tview, rtab, ent_out, rel_out,
               eidx_v, epair_v, ridx_v, rows_v, oct_v, rtab_v,
               sem, sem2, sem3):
    _sc_body(eidx, ridx, tview, rtab, ent_out, rel_out,
             eidx_v, epair_v, ridx_v, rows_v, oct_v, rtab_v,
             sem, sem2, sem3)


def kernel(idx, rel, ent_table, rel_table):
    eidx = jnp.transpose(idx.reshape(BATCH, 2)).reshape(ENT_N)
    eidx = eidx.astype(jnp.int32)
    ridx = rel.astype(jnp.int32)
    ttab = jnp.transpose(ent_table)
    tview = _pack_rows(ttab, ttab)
    rtab = jnp.transpose(rel_table).reshape(D * NUM_REL)
    ent_o, rel_o = _sc_lookup(eidx, ridx, tview, rtab)
    ent_emb = jnp.transpose(
        ent_o.reshape(2, ROCT, BATCH // CHUNK, 8, CHUNK),
        (2, 4, 0, 1, 3)).reshape(BATCH, 1, 2, D)
    rel_emb = jnp.transpose(
        rel_o.reshape(ROCT, BATCH // CHUNK, 8, CHUNK),
        (1, 3, 0, 2)).reshape(BATCH, D)
    return (ent_emb, rel_emb)


# speedup vs baseline: 2.1572x; 1.0463x over previous
import functools

import jax
import jax.numpy as jnp
from jax import lax
from jax.experimental import pallas as pl
from jax.experimental.pallas import tpu as pltpu
from jax.experimental.pallas import tpu_sc as plsc

D = 64
DP = 128
NUM_ENT = 1000000
NUM_REL = 1000
BATCH = 16384
NUM_WORKERS = 32
PER_W = BATCH // NUM_WORKERS
CHUNK = 128
LANES = 16
NBLK = PER_W // CHUNK
ROCT = D // 8
ENT_N = 2 * BATCH
OCT_W = 8 * CHUNK
TBLK = 16384
TGRID = 31
KP = TGRID * TBLK


def _transpose_block(lo_ref, hi_ref, out_ref):
    eye = (lax.broadcasted_iota(jnp.int32, (D, D), 0)
           == lax.broadcasted_iota(jnp.int32, (D, D), 1)).astype(jnp.float32)
    out_ref[:, 0:D] = lax.dot_general(
        lo_ref[...], eye, (((0,), (0,)), ((), ())),
        preferred_element_type=jnp.float32)
    out_ref[:, D:DP] = lax.dot_general(
        hi_ref[...], eye, (((0,), (0,)), ((), ())),
        preferred_element_type=jnp.float32)


_pack_rows = pl.pallas_call(
    _transpose_block,
    grid=(TGRID,),
    in_specs=[pl.BlockSpec((D, TBLK), lambda k: (0, k)),
              pl.BlockSpec((D, TBLK),
                           lambda k: (0, jnp.minimum(k + TGRID,
                                                     NUM_ENT // TBLK)))],
    out_specs=pl.BlockSpec((TBLK, DP), lambda k: (k, 0)),
    out_shape=jax.ShapeDtypeStruct((KP, DP), jnp.float32),
)


def _sc_body(eidx_hbm, ridx_hbm, tview_hbm, rtab_hbm,
             ent_out_hbm, rel_out_hbm,
             eidx_v, epair_v, ridx_v, rows_v, oct_v, rtab_v,
             sem, sem2, sem3):
    wid = lax.axis_index("s") * 2 + lax.axis_index("c")
    base = wid * PER_W

    rtab_copy = pltpu.async_copy(rtab_hbm, rtab_v, sem2)
    pltpu.sync_copy(eidx_hbm.at[pl.ds(base, PER_W)],
                    eidx_v.at[pl.ds(0, PER_W)])
    pltpu.sync_copy(eidx_hbm.at[pl.ds(BATCH + base, PER_W)],
                    eidx_v.at[pl.ds(PER_W, PER_W)])
    pltpu.sync_copy(ridx_hbm.at[pl.ds(base, PER_W)], ridx_v)

    lane_iota = lax.iota(jnp.int32, LANES)

    def ent_chunk(c, carry):
        j = c // NBLK
        ib = wid * NBLK + (c % NBLK)
        for g in range(CHUNK // LANES):
            evec = eidx_v[pl.ds(c * CHUNK + g * LANES, LANES)]
            hvec = jnp.where(evec >= KP, 1, 0)
            epair_v[pl.ds(g * LANES, LANES)] = evec - hvec * KP
        pltpu.async_copy(tview_hbm.at[epair_v], rows_v, sem).wait()

        def per_oct(r, carry2):
            rb = r & 1

            @pl.when(c * ROCT + r > 1)
            def _():
                pltpu.make_async_copy(oct_v.at[pl.ds(0, OCT_W)],
                                      ent_out_hbm.at[pl.ds(0, OCT_W)],
                                      sem3).wait()
            for g in range(CHUNK // LANES):
                evec = eidx_v[pl.ds(c * CHUNK + g * LANES, LANES)]
                rowvec = g * LANES + lane_iota
                colbase = jnp.where(evec >= KP, D, 0) + r * 8
                for dr in range(8):
                    vals = plsc.load_gather(rows_v, [rowvec, colbase + dr])
                    oct_v[pl.ds(rb * OCT_W + dr * CHUNK + g * LANES,
                                LANES)] = vals
            pltpu.async_copy(
                oct_v.at[pl.ds(rb * OCT_W, OCT_W)],
                ent_out_hbm.at[pl.ds(((j * ROCT + r) * (BATCH // CHUNK)
                                      + ib) * OCT_W, OCT_W)], sem3)
            return carry2

        lax.fori_loop(0, ROCT, per_oct, 0, unroll=False)
        return carry

    lax.fori_loop(0, 2 * NBLK, ent_chunk, 0, unroll=False)
    for _ in range(2):
        pltpu.make_async_copy(oct_v.at[pl.ds(0, OCT_W)],
                              ent_out_hbm.at[pl.ds(0, OCT_W)], sem3).wait()

    rtab_copy.wait()

    def rel_blk(b, carry):
        ib = wid * NBLK + b

        def per_oct(r, carry2):
            rb = r & 1

            @pl.when(b * ROCT + r > 1)
            def _():
                pltpu.make_async_copy(oct_v.at[pl.ds(0, OCT_W)],
                                      rel_out_hbm.at[pl.ds(0, OCT_W)],
                                      sem3).wait()
            for g in range(CHUNK // LANES):
                rvec = ridx_v[pl.ds(b * CHUNK + g * LANES, LANES)]
                for dr in range(8):
                    vals = plsc.load_gather(
                        rtab_v, [(r * 8 + dr) * NUM_REL + rvec])
                    oct_v[pl.ds(rb * OCT_W + dr * CHUNK + g * LANES,
                                LANES)] = vals
            pltpu.async_copy(
                oct_v.at[pl.ds(rb * OCT_W, OCT_W)],
                rel_out_hbm.at[pl.ds((r * (BATCH // CHUNK) + ib) * OCT_W,
                                     OCT_W)], sem3)
            return carry2

        lax.fori_loop(0, ROCT, per_oct, 0, unroll=False)
        return carry

    lax.fori_loop(0, NBLK, rel_blk, 0, unroll=False)
    for _ in range(2):
        pltpu.make_async_copy(oct_v.at[pl.ds(0, OCT_W)],
                              rel_out_hbm.at[pl.ds(0, OCT_W)], sem3).wait()


@functools.partial(
    pl.kernel,
    out_type=(
        jax.ShapeDtypeStruct((2 * D * BATCH,), jnp.float32),
        jax.ShapeDtypeStruct((D * BATCH,), jnp.float32),
    ),
    mesh=plsc.VectorSubcoreMesh(core_axis_name="c", subcore_axis_name="s"),
    compiler_params=pltpu.CompilerParams(use_tc_tiling_on_sc=False,
                                         needs_layout_passes=False),
    scratch_types=[
        pltpu.VMEM((2 * PER_W,), jnp.int32),
        pltpu.VMEM((CHUNK,), jnp.int32),
        pltpu.VMEM((PER_W,), jnp.int32),
        pltpu.VMEM((CHUNK, DP), jnp.float32),
        pltpu.VMEM((2 * OCT_W,), jnp.float32),
        pltpu.VMEM((D * NUM_REL,), jnp.float32),
        pltpu.SemaphoreType.DMA,
        pltpu.SemaphoreType.DMA,
        pltpu.SemaphoreType.DMA,
    ],
)
def _sc_lookup(eidx, ridx, tview, rtab, ent_out, rel_out,
               eidx_v, epair_v, ridx_v, rows_v, oct_v, rtab_v,
               sem, sem2, sem3):
    _sc_body(eidx, ridx, tview, rtab, ent_out, rel_out,
             eidx_v, epair_v, ridx_v, rows_v, oct_v, rtab_v,
             sem, sem2, sem3)


def kernel(idx, rel, ent_table, rel_table):
    eidx = jnp.transpose(idx.reshape(BATCH, 2)).reshape(ENT_N)
    eidx = eidx.astype(jnp.int32)
    ridx = rel.astype(jnp.int32)
    ttab = jnp.transpose(ent_table)
    tview = _pack_rows(ttab, ttab)
    rtab = jnp.transpose(rel_table).reshape(D * NUM_REL)
    ent_o, rel_o = _sc_lookup(eidx, ridx, tview, rtab)
    ent_emb = jnp.transpose(
        ent_o.reshape(2, ROCT, BATCH // CHUNK, 8, CHUNK),
        (2, 4, 0, 1, 3)).reshape(BATCH, 1, 2, D)
    rel_emb = jnp.transpose(
        rel_o.reshape(ROCT, BATCH // CHUNK, 8, CHUNK),
        (1, 3, 0, 2)).reshape(BATCH, D)
    return (ent_emb, rel_emb)


# speedup vs baseline: 2.2244x; 1.0311x over previous
import functools

import jax
import jax.numpy as jnp
from jax import lax
from jax.experimental import pallas as pl
from jax.experimental.pallas import tpu as pltpu
from jax.experimental.pallas import tpu_sc as plsc

D = 64
DP = 128
NUM_ENT = 1000000
NUM_REL = 1000
BATCH = 16384
NUM_WORKERS = 32
PER_W = BATCH // NUM_WORKERS
CHUNK = 128
LANES = 16
NBLK = PER_W // CHUNK
ROCT = D // 8
ENT_N = 2 * BATCH
OCT_W = 8 * CHUNK
TBLK = 16384
TGRID = 31
KP = TGRID * TBLK


def _transpose_block(lo_ref, hi_ref, out_ref):
    eye = (lax.broadcasted_iota(jnp.int32, (D, D), 0)
           == lax.broadcasted_iota(jnp.int32, (D, D), 1)).astype(jnp.float32)
    out_ref[:, 0:D] = lax.dot_general(
        lo_ref[...], eye, (((0,), (0,)), ((), ())),
        preferred_element_type=jnp.float32)
    out_ref[:, D:DP] = lax.dot_general(
        hi_ref[...], eye, (((0,), (0,)), ((), ())),
        preferred_element_type=jnp.float32)


_pack_rows = pl.pallas_call(
    _transpose_block,
    grid=(TGRID,),
    in_specs=[pl.BlockSpec((D, TBLK), lambda k: (0, k)),
              pl.BlockSpec((D, TBLK),
                           lambda k: (0, jnp.minimum(k + TGRID,
                                                     NUM_ENT // TBLK)))],
    out_specs=pl.BlockSpec((TBLK, DP), lambda k: (k, 0)),
    out_shape=jax.ShapeDtypeStruct((KP, DP), jnp.float32),
)


def _sc_body(eidx_hbm, ridx_hbm, tview_hbm, rtab_hbm,
             ent_out_hbm, rel_out_hbm,
             eidx_v, epair_v, ridx_v, rows_v, oct_v, rtab_v,
             sem, sem2, sem3):
    wid = lax.axis_index("s") * 2 + lax.axis_index("c")
    base = wid * PER_W

    rtab_copy = pltpu.async_copy(rtab_hbm, rtab_v, sem2)
    pltpu.sync_copy(eidx_hbm.at[pl.ds(base, PER_W)],
                    eidx_v.at[pl.ds(0, PER_W)])
    pltpu.sync_copy(eidx_hbm.at[pl.ds(BATCH + base, PER_W)],
                    eidx_v.at[pl.ds(PER_W, PER_W)])
    pltpu.sync_copy(ridx_hbm.at[pl.ds(base, PER_W)], ridx_v)

    lane_iota = lax.iota(jnp.int32, LANES)

    def compute_epair(c):
        for g in range(CHUNK // LANES):
            evec = eidx_v[pl.ds(c * CHUNK + g * LANES, LANES)]
            hvec = jnp.where(evec >= KP, 1, 0)
            epair_v[pl.ds((c & 1) * CHUNK + g * LANES, LANES)] = (
                evec - hvec * KP)

    def fire_gather(c):
        pltpu.async_copy(
            tview_hbm.at[epair_v.at[pl.ds((c & 1) * CHUNK, CHUNK)]],
            rows_v.at[pl.ds((c & 1) * CHUNK, CHUNK), :], sem)

    compute_epair(0)
    fire_gather(0)

    def ent_chunk(c, carry):
        j = c // NBLK
        ib = wid * NBLK + (c % NBLK)
        cb = c & 1
        pltpu.make_async_copy(
            tview_hbm.at[epair_v.at[pl.ds(cb * CHUNK, CHUNK)]],
            rows_v.at[pl.ds(cb * CHUNK, CHUNK), :], sem).wait()

        @pl.when(c < 2 * NBLK - 1)
        def _():
            compute_epair(c + 1)
            fire_gather(c + 1)

        def per_oct(r, carry2):
            rb = r & 1

            @pl.when(c * ROCT + r > 1)
            def _():
                pltpu.make_async_copy(oct_v.at[pl.ds(0, OCT_W)],
                                      ent_out_hbm.at[pl.ds(0, OCT_W)],
                                      sem3).wait()
            for g in range(CHUNK // LANES):
                evec = eidx_v[pl.ds(c * CHUNK + g * LANES, LANES)]
                rowvec = cb * CHUNK + g * LANES + lane_iota
                colbase = jnp.where(evec >= KP, D, 0) + r * 8
                for dr in range(8):
                    vals = plsc.load_gather(rows_v, [rowvec, colbase + dr])
                    oct_v[pl.ds(rb * OCT_W + dr * CHUNK + g * LANES,
                                LANES)] = vals
            pltpu.async_copy(
                oct_v.at[pl.ds(rb * OCT_W, OCT_W)],
                ent_out_hbm.at[pl.ds(((j * ROCT + r) * (BATCH // CHUNK)
                                      + ib) * OCT_W, OCT_W)], sem3)
            return carry2

        lax.fori_loop(0, ROCT, per_oct, 0, unroll=False)
        return carry

    lax.fori_loop(0, 2 * NBLK, ent_chunk, 0, unroll=False)
    for _ in range(2):
        pltpu.make_async_copy(oct_v.at[pl.ds(0, OCT_W)],
                              ent_out_hbm.at[pl.ds(0, OCT_W)], sem3).wait()

    rtab_copy.wait()

    def rel_blk(b, carry):
        ib = wid * NBLK + b

        def per_oct(r, carry2):
            rb = r & 1

            @pl.when(b * ROCT + r > 1)
            def _():
                pltpu.make_async_copy(oct_v.at[pl.ds(0, OCT_W)],
                                      rel_out_hbm.at[pl.ds(0, OCT_W)],
                                      sem3).wait()
            for g in range(CHUNK // LANES):
                rvec = ridx_v[pl.ds(b * CHUNK + g * LANES, LANES)]
                for dr in range(8):
                    vals = plsc.load_gather(
                        rtab_v, [(r * 8 + dr) * NUM_REL + rvec])
                    oct_v[pl.ds(rb * OCT_W + dr * CHUNK + g * LANES,
                                LANES)] = vals
            pltpu.async_copy(
                oct_v.at[pl.ds(rb * OCT_W, OCT_W)],
                rel_out_hbm.at[pl.ds((r * (BATCH // CHUNK) + ib) * OCT_W,
                                     OCT_W)], sem3)
            return carry2

        lax.fori_loop(0, ROCT, per_oct, 0, unroll=False)
        return carry

    lax.fori_loop(0, NBLK, rel_blk, 0, unroll=False)
    for _ in range(2):
        pltpu.make_async_copy(oct_v.at[pl.ds(0, OCT_W)],
                              rel_out_hbm.at[pl.ds(0, OCT_W)], sem3).wait()


@functools.partial(
    pl.kernel,
    out_type=(
        jax.ShapeDtypeStruct((2 * D * BATCH,), jnp.float32),
        jax.ShapeDtypeStruct((D * BATCH,), jnp.float32),
    ),
    mesh=plsc.VectorSubcoreMesh(core_axis_name="c", subcore_axis_name="s"),
    compiler_params=pltpu.CompilerParams(use_tc_tiling_on_sc=False,
                                         needs_layout_passes=False),
    scratch_types=[
        pltpu.VMEM((2 * PER_W,), jnp.int32),
        pltpu.VMEM((2 * CHUNK,), jnp.int32),
        pltpu.VMEM((PER_W,), jnp.int32),
        pltpu.VMEM((2 * CHUNK, DP), jnp.float32),
        pltpu.VMEM((2 * OCT_W,), jnp.float32),
        pltpu.VMEM((D * NUM_REL,), jnp.float32),
        pltpu.SemaphoreType.DMA,
        pltpu.SemaphoreType.DMA,
        pltpu.SemaphoreType.DMA,
    ],
)
def _sc_lookup(eidx, ridx, tview, rtab, ent_out, rel_out,
               eidx_v, epair_v, ridx_v, rows_v, oct_v, rtab_v,
               sem, sem2, sem3):
    _sc_body(eidx, ridx, tview, rtab, ent_out, rel_out,
             eidx_v, epair_v, ridx_v, rows_v, oct_v, rtab_v,
             sem, sem2, sem3)


def kernel(idx, rel, ent_table, rel_table):
    eidx = jnp.transpose(idx.reshape(BATCH, 2)).reshape(ENT_N)
    eidx = eidx.astype(jnp.int32)
    ridx = rel.astype(jnp.int32)
    ttab = jnp.transpose(ent_table)
    tview = _pack_rows(ttab, ttab)
    rtab = jnp.transpose(rel_table).reshape(D * NUM_REL)
    ent_o, rel_o = _sc_lookup(eidx, ridx, tview, rtab)
    ent_emb = jnp.transpose(
        ent_o.reshape(2, ROCT, BATCH // CHUNK, 8, CHUNK),
        (2, 4, 0, 1, 3)).reshape(BATCH, 1, 2, D)
    rel_emb = jnp.transpose(
        rel_o.reshape(ROCT, BATCH // CHUNK, 8, CHUNK),
        (1, 3, 0, 2)).reshape(BATCH, D)
    return (ent_emb, rel_emb)
